# Initial kernel scaffold; baseline (speedup 1.0000x reference)
import jax
import jax.numpy as jnp
from jax.experimental import pallas as pl


def kernel(x, edge_index, edge_attr, batch, lin_w, lin_b, fc2_w, fc2_b, fco_w, fco_b):
    raise NotImplementedError("write your pallas kernel here")


# speedup vs baseline: 11.7079x; 11.7079x over previous
import dataclasses
import functools

import jax
import jax.numpy as jnp
from jax import lax
from jax.experimental import pallas as pl
from jax.experimental.pallas import tpu as pltpu
from jax.experimental.pallas import tpu_sc as plsc

N = 10000
E = 320000
D = 128
OUT = 64
G = 64

NC, NS, L = 2, 16, 16
NW = NC * NS
CH = 128
CPT = (-(-E // (NW * CH)) + 7) // 8 * 8
E_PAD = NW * CH * CPT
NCH = E_PAD // CH
R = 10240
STRIPE = R // NS
NP = R
BLK = 2048
NB = NP // BLK

def _deg_body(row_hbm, out_hbm, idx_v, dacc):
    ci = lax.axis_index("c")
    si = lax.axis_index("s")
    wid = si * NC + ci

    @pl.loop(0, R, step=L)
    def _(k):
        dacc[pl.ds(k, L)] = jnp.zeros((L,), jnp.float32)

    pltpu.sync_copy(row_hbm.at[pl.ds(wid * CPT, CPT)], idx_v)

    ones = jnp.ones((L,), jnp.float32)

    @pl.loop(0, CPT)
    def _(j):
        for t in range(CH // L):
            idxv = idx_v[j, pl.ds(t * L, L)]
            plsc.addupdate_scatter(dacc, [idxv], ones)

    pltpu.sync_copy(dacc, out_hbm.at[wid, 0])


def _agg_body(y_hbm, row_hbm, col_hbm, ea_hbm, out_hbm, row_v, col_v, ea_v, rb, acc):
    ci = lax.axis_index("c")
    si = lax.axis_index("s")
    wid = si * NC + ci

    @pl.loop(0, CH)
    def _(r):
        for d in range(D // L):
            rb[r, pl.ds(d * L, L)] = jnp.zeros((L,), jnp.float32)

    @pl.loop(0, STRIPE // CH)
    def _(k):
        pltpu.sync_copy(rb, acc.at[pl.ds(si * STRIPE + k * CH, CH)])

    plsc.subcore_barrier()

    pltpu.sync_copy(row_hbm.at[pl.ds(wid * CPT, CPT)], row_v)
    pltpu.sync_copy(col_hbm.at[pl.ds(wid * CPT, CPT)], col_v)
    pltpu.sync_copy(ea_hbm.at[pl.ds(wid * CPT, CPT)], ea_v)

    @pl.loop(0, CPT)
    def _(j):
        pltpu.sync_copy(y_hbm.at[row_v.at[j]], rb)

        @pl.loop(0, CH, step=L)
        def _(e0):
            ev = ea_v[j, pl.ds(e0, L)]
            for t in range(L):
                s = ev[t]
                for d in range(D // L):
                    sl = pl.ds(d * L, L)
                    rb[e0 + t, sl] = rb[e0 + t, sl] * s

        pltpu.sync_copy(rb, acc.at[col_v.at[j]], add=True)

    plsc.subcore_barrier()

    @pl.loop(0, STRIPE // CH)
    def _(k):
        base = si * STRIPE + k * CH
        pltpu.sync_copy(acc.at[pl.ds(base, CH)], out_hbm.at[ci, pl.ds(base, CH)])


@functools.cache
def _sc_kernels():
    mesh = plsc.VectorSubcoreMesh(core_axis_name="c", subcore_axis_name="s",
                                  num_cores=NC, num_subcores=NS)
    cp = pltpu.CompilerParams()
    if "needs_layout_passes" in pltpu.CompilerParams.__dataclass_fields__:
        cp = dataclasses.replace(cp, needs_layout_passes=False)
    deg_kernel = pl.kernel(
        _deg_body,
        out_type=jax.ShapeDtypeStruct((NW, 1, R), jnp.float32),
        mesh=mesh,
        compiler_params=cp,
        scratch_types=[
            pltpu.VMEM((CPT, CH), jnp.int32),
            pltpu.VMEM((R,), jnp.float32),
        ],
    )
    agg_kernel = pl.kernel(
        _agg_body,
        out_type=jax.ShapeDtypeStruct((NC, R, D), jnp.float32),
        mesh=mesh,
        scratch_types=[
            pltpu.VMEM((CPT, CH), jnp.int32),
            pltpu.VMEM((CPT, CH), jnp.int32),
            pltpu.VMEM((CPT, CH), jnp.float32),
            pltpu.VMEM((CH, D), jnp.float32),
            pltpu.VMEM_SHARED((R, D), jnp.float32),
        ],
    )
    return deg_kernel, agg_kernel


def _deg_col(cnt2):
    return lax.dot_general(cnt2, jnp.ones((NW, 1), jnp.float32),
                           (((0,), (0,)), ((), ())),
                           precision=lax.Precision.HIGHEST,
                           preferred_element_type=jnp.float32)


def _prep_body(cnt_ref, x_ref, y_ref):
    deg = 1.0 + _deg_col(cnt_ref[...])
    dis = lax.rsqrt(deg)
    y_ref[...] = x_ref[...] * dis


def _final_body(p0, p1, cnt, y, lw, lb, fw, fb, b2, ow, owb, out_ref,
                opt_ref, gs, cn):
    dn = (((1,), (1,)), ((), ()))
    dt = (((0,), (0,)), ((), ()))
    hi = lax.Precision.HIGHEST
    i = pl.program_id(0)

    deg = 1.0 + _deg_col(cnt[...])
    dis = lax.rsqrt(deg)
    aggr = dis * (p0[...] + p1[...] + y[...])
    h = jnp.tanh(lax.dot_general(aggr, lw[...], dn, precision=hi,
                                 preferred_element_type=jnp.float32) + lb[...])
    out_ref[...] = jax.nn.sigmoid(
        lax.dot_general(h, fw[...], dn, precision=hi,
                        preferred_element_type=jnp.float32) + fb[...])

    grp = b2[...]
    iota = lax.broadcasted_iota(jnp.int32, (BLK, 128), 1)
    onehot = (grp == iota).astype(jnp.float32)
    gsum = lax.dot_general(onehot, h, dt, precision=hi,
                           preferred_element_type=jnp.float32)
    cnt = lax.dot_general(onehot, jnp.ones((BLK, 128), jnp.float32), dt,
                          precision=hi, preferred_element_type=jnp.float32)

    @pl.when(i == 0)
    def _():
        gs[...] = jnp.zeros_like(gs)
        cn[...] = jnp.zeros_like(cn)

    gs[...] += gsum
    cn[...] += cnt

    @pl.when(i == NB - 1)
    def _():
        gemb = gs[...][:G, :] / jnp.maximum(cn[...][:G, :], 1.0)
        opt_ref[...] = jax.nn.sigmoid(
            lax.dot_general(gemb, ow[...], (((1,), (0,)), ((), ())),
                            precision=hi,
                            preferred_element_type=jnp.float32) + owb[...])


def kernel(x, edge_index, edge_attr, batch, lin_w, lin_b, fc2_w, fc2_b, fco_w, fco_b):
    row = edge_index[0]
    col = edge_index[1]
    pad_e = E_PAD - E
    rowp = jnp.concatenate([row, jnp.full((pad_e,), N, row.dtype)]).reshape(NCH, CH)
    colp = jnp.concatenate([col, jnp.full((pad_e,), N, col.dtype)]).reshape(NCH, CH)
    eap = jnp.concatenate(
        [edge_attr, jnp.zeros((pad_e,), edge_attr.dtype)]).reshape(NCH, CH)
    xp = jnp.concatenate([x, jnp.zeros((NP - N, D), x.dtype)])
    b2 = jnp.concatenate([batch, jnp.full((NP - N,), 127, batch.dtype)]).reshape(NP, 1)

    deg_kernel, agg_kernel = _sc_kernels()
    cnt2 = deg_kernel(rowp).reshape(NW, R)

    y = pl.pallas_call(
        _prep_body,
        out_shape=jax.ShapeDtypeStruct((NP, D), jnp.float32),
    )(cnt2, xp)

    parts = agg_kernel(y, rowp, colp, eap)
    p0, p1 = parts[0], parts[1]

    ow = jnp.zeros((D, 128), jnp.float32).at[:, 0].set(fco_w[0])
    owb = jnp.zeros((1, 128), jnp.float32).at[0, 0].set(fco_b[0])
    full = lambda shape: pl.BlockSpec(shape, lambda i: (0, 0))
    outp, opt = pl.pallas_call(
        _final_body,
        grid=(NB,),
        in_specs=[
            pl.BlockSpec((BLK, D), lambda i: (i, 0)),
            pl.BlockSpec((BLK, D), lambda i: (i, 0)),
            pl.BlockSpec((NW, BLK), lambda i: (0, i)),
            pl.BlockSpec((BLK, D), lambda i: (i, 0)),
            full((D, D)), full((1, D)),
            full((OUT, D)), full((1, OUT)),
            pl.BlockSpec((BLK, 1), lambda i: (i, 0)),
            full((D, 128)), full((1, 128)),
        ],
        out_specs=(pl.BlockSpec((BLK, OUT), lambda i: (i, 0)),
                   pl.BlockSpec((G, 128), lambda i: (0, 0))),
        out_shape=(jax.ShapeDtypeStruct((NP, OUT), jnp.float32),
                   jax.ShapeDtypeStruct((G, 128), jnp.float32)),
        scratch_shapes=[pltpu.VMEM((128, D), jnp.float32),
                        pltpu.VMEM((128, 128), jnp.float32)],
    )(p0, p1, cnt2, y, lin_w, lin_b.reshape(1, D), fc2_w,
      fc2_b.reshape(1, OUT), b2, ow, owb)

    return (outp[:N], opt[:, 0:1])


# speedup vs baseline: 13.2486x; 1.1316x over previous
import dataclasses
import functools

import jax
import jax.numpy as jnp
from jax import lax
from jax.experimental import pallas as pl
from jax.experimental.pallas import tpu as pltpu
from jax.experimental.pallas import tpu_sc as plsc

N = 10000
E = 320000
D = 128
OUT = 64
G = 64

NC, NS, L = 2, 16, 16
NW = NC * NS
CH = 128
CPT = (-(-E // (NW * CH)) + 7) // 8 * 8
E_PAD = NW * CH * CPT
NCH = E_PAD // CH
NSEG = 2
SPT = CPT // NSEG
R = 10240
STRIPE = R // NS
NP = R
BLK = 2048
NB = NP // BLK

def _deg_body(row_hbm, out_hbm, idx_v, dacc):
    ci = lax.axis_index("c")
    si = lax.axis_index("s")
    wid = si * NC + ci

    @pl.loop(0, R, step=L)
    def _(k):
        dacc[pl.ds(k, L)] = jnp.zeros((L,), jnp.float32)

    pltpu.sync_copy(row_hbm.at[pl.ds(wid * CPT, CPT)], idx_v)

    ones = jnp.ones((L,), jnp.float32)

    @pl.loop(0, CPT)
    def _(j):
        for t in range(CH // L):
            idxv = idx_v[j, pl.ds(t * L, L)]
            plsc.addupdate_scatter(dacc, [idxv], ones)

    pltpu.sync_copy(dacc, out_hbm.at[wid, 0])


def _agg_body(y_hbm, row_hbm, col_hbm, ea_hbm, out_hbm, row_v, col_v, ea_v,
              rb0, rb1, acc, g0, g1, s0, s1):
    ci = lax.axis_index("c")
    si = lax.axis_index("s")
    wid = si * NC + ci

    @pl.loop(0, CH)
    def _(r):
        for d in range(D // L):
            rb0[r, pl.ds(d * L, L)] = jnp.zeros((L,), jnp.float32)

    @pl.loop(0, STRIPE // CH)
    def _(k):
        pltpu.sync_copy(rb0, acc.at[pl.ds(si * STRIPE + k * CH, CH)])

    plsc.subcore_barrier()

    def scale(rb, j):
        @pl.loop(0, CH, step=L)
        def _(e0):
            ev = ea_v[j, pl.ds(e0, L)]
            for t in range(L):
                s = ev[t]
                for d in range(D // L):
                    sl = pl.ds(d * L, L)
                    rb[e0 + t, sl] = rb[e0 + t, sl] * s

    for seg in range(NSEG):
        base = wid * CPT + seg * SPT
        pltpu.sync_copy(row_hbm.at[pl.ds(base, SPT)], row_v)
        pltpu.sync_copy(col_hbm.at[pl.ds(base, SPT)], col_v)
        pltpu.sync_copy(ea_hbm.at[pl.ds(base, SPT)], ea_v)

        pltpu.async_copy(y_hbm.at[row_v.at[0]], rb0, g0)

        @pl.loop(0, SPT, step=2)
        def _(j0):
            pltpu.make_async_copy(y_hbm.at[row_v.at[j0]], rb0, g0).wait()

            @pl.when(j0 > 0)
            def _():
                pltpu.make_async_copy(rb1, acc.at[col_v.at[j0 - 1]], s1).wait()

            g1d = pltpu.async_copy(y_hbm.at[row_v.at[j0 + 1]], rb1, g1)
            scale(rb0, j0)
            s0d = pltpu.async_copy(rb0, acc.at[col_v.at[j0]], s0, add=True)

            g1d.wait()
            s0d.wait()

            @pl.when(j0 + 2 < SPT)
            def _():
                pltpu.async_copy(y_hbm.at[row_v.at[j0 + 2]], rb0, g0)

            scale(rb1, j0 + 1)
            pltpu.async_copy(rb1, acc.at[col_v.at[j0 + 1]], s1, add=True)

        pltpu.make_async_copy(rb1, acc.at[col_v.at[SPT - 1]], s1).wait()

    plsc.subcore_barrier()

    @pl.loop(0, STRIPE // CH)
    def _(k):
        base = si * STRIPE + k * CH
        pltpu.sync_copy(acc.at[pl.ds(base, CH)], out_hbm.at[ci, pl.ds(base, CH)])


@functools.cache
def _sc_kernels():
    mesh = plsc.VectorSubcoreMesh(core_axis_name="c", subcore_axis_name="s",
                                  num_cores=NC, num_subcores=NS)
    cp = pltpu.CompilerParams()
    if "needs_layout_passes" in pltpu.CompilerParams.__dataclass_fields__:
        cp = dataclasses.replace(cp, needs_layout_passes=False)
    deg_kernel = pl.kernel(
        _deg_body,
        out_type=jax.ShapeDtypeStruct((NW, 1, R), jnp.float32),
        mesh=mesh,
        compiler_params=cp,
        scratch_types=[
            pltpu.VMEM((CPT, CH), jnp.int32),
            pltpu.VMEM((R,), jnp.float32),
        ],
    )
    agg_kernel = pl.kernel(
        _agg_body,
        out_type=jax.ShapeDtypeStruct((NC, R, D), jnp.float32),
        mesh=mesh,
        scratch_types=[
            pltpu.VMEM((SPT, CH), jnp.int32),
            pltpu.VMEM((SPT, CH), jnp.int32),
            pltpu.VMEM((SPT, CH), jnp.float32),
            pltpu.VMEM((CH, D), jnp.float32),
            pltpu.VMEM((CH, D), jnp.float32),
            pltpu.VMEM_SHARED((R, D), jnp.float32),
            pltpu.SemaphoreType.DMA,
            pltpu.SemaphoreType.DMA,
            pltpu.SemaphoreType.DMA,
            pltpu.SemaphoreType.DMA,
        ],
    )
    return deg_kernel, agg_kernel


def _deg_col(cnt2):
    return lax.dot_general(cnt2, jnp.ones((NW, 1), jnp.float32),
                           (((0,), (0,)), ((), ())),
                           precision=lax.Precision.HIGHEST,
                           preferred_element_type=jnp.float32)


def _prep_body(cnt_ref, x_ref, y_ref):
    deg = 1.0 + _deg_col(cnt_ref[...])
    dis = lax.rsqrt(deg)
    y_ref[...] = x_ref[...] * dis


def _final_body(p0, p1, cnt, y, lw, lb, fw, fb, b2, ow, owb, out_ref,
                opt_ref, gs, cn):
    dn = (((1,), (1,)), ((), ()))
    dt = (((0,), (0,)), ((), ()))
    hi = lax.Precision.HIGHEST
    i = pl.program_id(0)

    deg = 1.0 + _deg_col(cnt[...])
    dis = lax.rsqrt(deg)
    aggr = dis * (p0[...] + p1[...] + y[...])
    h = jnp.tanh(lax.dot_general(aggr, lw[...], dn, precision=hi,
                                 preferred_element_type=jnp.float32) + lb[...])
    out_ref[...] = jax.nn.sigmoid(
        lax.dot_general(h, fw[...], dn, precision=hi,
                        preferred_element_type=jnp.float32) + fb[...])

    grp = b2[...]
    iota = lax.broadcasted_iota(jnp.int32, (BLK, 128), 1)
    onehot = (grp == iota).astype(jnp.float32)
    gsum = lax.dot_general(onehot, h, dt, precision=hi,
                           preferred_element_type=jnp.float32)
    cnt = lax.dot_general(onehot, jnp.ones((BLK, 128), jnp.float32), dt,
                          precision=hi, preferred_element_type=jnp.float32)

    @pl.when(i == 0)
    def _():
        gs[...] = jnp.zeros_like(gs)
        cn[...] = jnp.zeros_like(cn)

    gs[...] += gsum
    cn[...] += cnt

    @pl.when(i == NB - 1)
    def _():
        gemb = gs[...][:G, :] / jnp.maximum(cn[...][:G, :], 1.0)
        opt_ref[...] = jax.nn.sigmoid(
            lax.dot_general(gemb, ow[...], (((1,), (0,)), ((), ())),
                            precision=hi,
                            preferred_element_type=jnp.float32) + owb[...])


def kernel(x, edge_index, edge_attr, batch, lin_w, lin_b, fc2_w, fc2_b, fco_w, fco_b):
    row = edge_index[0]
    col = edge_index[1]
    pad_e = E_PAD - E
    rowp = jnp.concatenate([row, jnp.full((pad_e,), N, row.dtype)]).reshape(NCH, CH)
    colp = jnp.concatenate([col, jnp.full((pad_e,), N, col.dtype)]).reshape(NCH, CH)
    eap = jnp.concatenate(
        [edge_attr, jnp.zeros((pad_e,), edge_attr.dtype)]).reshape(NCH, CH)
    xp = jnp.concatenate([x, jnp.zeros((NP - N, D), x.dtype)])
    b2 = jnp.concatenate([batch, jnp.full((NP - N,), 127, batch.dtype)]).reshape(NP, 1)

    deg_kernel, agg_kernel = _sc_kernels()
    cnt2 = deg_kernel(rowp).reshape(NW, R)

    y = pl.pallas_call(
        _prep_body,
        out_shape=jax.ShapeDtypeStruct((NP, D), jnp.float32),
    )(cnt2, xp)

    parts = agg_kernel(y, rowp, colp, eap)
    p0, p1 = parts[0], parts[1]

    ow = jnp.zeros((D, 128), jnp.float32).at[:, 0].set(fco_w[0])
    owb = jnp.zeros((1, 128), jnp.float32).at[0, 0].set(fco_b[0])
    full = lambda shape: pl.BlockSpec(shape, lambda i: (0, 0))
    outp, opt = pl.pallas_call(
        _final_body,
        grid=(NB,),
        in_specs=[
            pl.BlockSpec((BLK, D), lambda i: (i, 0)),
            pl.BlockSpec((BLK, D), lambda i: (i, 0)),
            pl.BlockSpec((NW, BLK), lambda i: (0, i)),
            pl.BlockSpec((BLK, D), lambda i: (i, 0)),
            full((D, D)), full((1, D)),
            full((OUT, D)), full((1, OUT)),
            pl.BlockSpec((BLK, 1), lambda i: (i, 0)),
            full((D, 128)), full((1, 128)),
        ],
        out_specs=(pl.BlockSpec((BLK, OUT), lambda i: (i, 0)),
                   pl.BlockSpec((G, 128), lambda i: (0, 0))),
        out_shape=(jax.ShapeDtypeStruct((NP, OUT), jnp.float32),
                   jax.ShapeDtypeStruct((G, 128), jnp.float32)),
        scratch_shapes=[pltpu.VMEM((128, D), jnp.float32),
                        pltpu.VMEM((128, 128), jnp.float32)],
    )(p0, p1, cnt2, y, lin_w, lin_b.reshape(1, D), fc2_w,
      fc2_b.reshape(1, OUT), b2, ow, owb)

    return (outp[:N], opt[:, 0:1])


# speedup vs baseline: 15.7501x; 1.1888x over previous
import dataclasses
import functools

import jax
import jax.numpy as jnp
from jax import lax
from jax.experimental import pallas as pl
from jax.experimental.pallas import tpu as pltpu
from jax.experimental.pallas import tpu_sc as plsc

N = 10000
E = 320000
D = 128
OUT = 64
G = 64

NC, NS, L = 2, 16, 16
NW = NC * NS
CH = 128
CPT = (-(-E // (NW * CH)) + 7) // 8 * 8
E_PAD = NW * CH * CPT
NCH = E_PAD // CH
SPT = 40
TOT = 2 * CPT
CPT0 = 120
CPT1 = TOT - CPT0
R = 10240
STRIPE = R // NS
NP = R
BLK = 2048
NB = NP // BLK

def _deg_body(row_hbm, out_hbm, idx_v, dacc):
    ci = lax.axis_index("c")
    si = lax.axis_index("s")
    wid = si * NC + ci

    @pl.loop(0, R, step=L)
    def _(k):
        dacc[pl.ds(k, L)] = jnp.zeros((L,), jnp.float32)

    pltpu.sync_copy(row_hbm.at[pl.ds(wid * CPT, CPT)], idx_v)

    ones = jnp.ones((L,), jnp.float32)

    @pl.loop(0, CPT)
    def _(j):
        for t in range(CH // L):
            idxv = idx_v[j, pl.ds(t * L, L)]
            plsc.addupdate_scatter(dacc, [idxv], ones)

    pltpu.sync_copy(dacc, out_hbm.at[wid, 0])


def _agg_body(y_hbm, row_hbm, col_hbm, ea_hbm, out_hbm, row_v, col_v, ea_v,
              rb0, rb1, acc, g0, g1, s0, s1):
    ci = lax.axis_index("c")
    si = lax.axis_index("s")
    wid = si * NC + ci

    @pl.loop(0, CH)
    def _(r):
        for d in range(D // L):
            rb0[r, pl.ds(d * L, L)] = jnp.zeros((L,), jnp.float32)

    @pl.loop(0, STRIPE // CH)
    def _(k):
        pltpu.sync_copy(rb0, acc.at[pl.ds(si * STRIPE + k * CH, CH)])

    plsc.subcore_barrier()

    def scale(rb, j):
        @pl.loop(0, CH, step=L)
        def _(e0):
            ev = ea_v[j, pl.ds(e0, L)]
            for t in range(L):
                s = ev[t]
                for d in range(D // L):
                    sl = pl.ds(d * L, L)
                    rb[e0 + t, sl] = rb[e0 + t, sl] * s

    my_base = si * TOT + ci * CPT0
    my_nseg = lax.select(ci == 0, CPT0 // SPT, CPT1 // SPT)
    for seg in range(TOT // SPT):
        @pl.when(seg < my_nseg)
        def _():
            base = my_base + seg * SPT
            pltpu.sync_copy(row_hbm.at[pl.ds(base, SPT)], row_v)
            pltpu.sync_copy(col_hbm.at[pl.ds(base, SPT)], col_v)
            pltpu.sync_copy(ea_hbm.at[pl.ds(base, SPT)], ea_v)

            pltpu.async_copy(y_hbm.at[row_v.at[0]], rb0, g0)

            @pl.loop(0, SPT, step=2)
            def _(j0):
                pltpu.make_async_copy(y_hbm.at[row_v.at[j0]], rb0, g0).wait()

                @pl.when(j0 > 0)
                def _():
                    pltpu.make_async_copy(
                        rb1, acc.at[col_v.at[j0 - 1]], s1).wait()

                g1d = pltpu.async_copy(y_hbm.at[row_v.at[j0 + 1]], rb1, g1)
                scale(rb0, j0)
                s0d = pltpu.async_copy(rb0, acc.at[col_v.at[j0]], s0, add=True)

                g1d.wait()
                s0d.wait()

                @pl.when(j0 + 2 < SPT)
                def _():
                    pltpu.async_copy(y_hbm.at[row_v.at[j0 + 2]], rb0, g0)

                scale(rb1, j0 + 1)
                pltpu.async_copy(rb1, acc.at[col_v.at[j0 + 1]], s1, add=True)

            pltpu.make_async_copy(rb1, acc.at[col_v.at[SPT - 1]], s1).wait()

    plsc.subcore_barrier()

    @pl.loop(0, STRIPE // CH)
    def _(k):
        base = si * STRIPE + k * CH
        pltpu.sync_copy(acc.at[pl.ds(base, CH)], out_hbm.at[ci, pl.ds(base, CH)])


@functools.cache
def _sc_kernels():
    mesh = plsc.VectorSubcoreMesh(core_axis_name="c", subcore_axis_name="s",
                                  num_cores=NC, num_subcores=NS)
    cp = pltpu.CompilerParams()
    if "needs_layout_passes" in pltpu.CompilerParams.__dataclass_fields__:
        cp = dataclasses.replace(cp, needs_layout_passes=False)
    deg_kernel = pl.kernel(
        _deg_body,
        out_type=jax.ShapeDtypeStruct((NW, 1, R), jnp.float32),
        mesh=mesh,
        compiler_params=cp,
        scratch_types=[
            pltpu.VMEM((CPT, CH), jnp.int32),
            pltpu.VMEM((R,), jnp.float32),
        ],
    )
    agg_kernel = pl.kernel(
        _agg_body,
        out_type=jax.ShapeDtypeStruct((NC, R, D), jnp.float32),
        mesh=mesh,
        scratch_types=[
            pltpu.VMEM((SPT, CH), jnp.int32),
            pltpu.VMEM((SPT, CH), jnp.int32),
            pltpu.VMEM((SPT, CH), jnp.float32),
            pltpu.VMEM((CH, D), jnp.float32),
            pltpu.VMEM((CH, D), jnp.float32),
            pltpu.VMEM_SHARED((R, D), jnp.float32),
            pltpu.SemaphoreType.DMA,
            pltpu.SemaphoreType.DMA,
            pltpu.SemaphoreType.DMA,
            pltpu.SemaphoreType.DMA,
        ],
    )
    return deg_kernel, agg_kernel


def _deg_col(cnt2):
    return lax.dot_general(cnt2, jnp.ones((NW, 1), jnp.float32),
                           (((0,), (0,)), ((), ())),
                           precision=lax.Precision.HIGHEST,
                           preferred_element_type=jnp.float32)


def _prep_body(cnt_ref, x_ref, y_ref):
    deg = 1.0 + _deg_col(cnt_ref[...])
    dis = lax.rsqrt(deg)
    y_ref[...] = x_ref[...] * dis


def _final_body(p0, p1, cnt, y, lw, lb, fw, fb, b2, ow, owb, out_ref,
                opt_ref, gs, cn):
    dn = (((1,), (1,)), ((), ()))
    dt = (((0,), (0,)), ((), ()))
    hi = lax.Precision.HIGHEST
    i = pl.program_id(0)

    deg = 1.0 + _deg_col(cnt[...])
    dis = lax.rsqrt(deg)
    aggr = dis * (p0[...] + p1[...] + y[...])
    h = jnp.tanh(lax.dot_general(aggr, lw[...], dn, precision=hi,
                                 preferred_element_type=jnp.float32) + lb[...])
    out_ref[...] = jax.nn.sigmoid(
        lax.dot_general(h, fw[...], dn, precision=hi,
                        preferred_element_type=jnp.float32) + fb[...])

    grp = b2[...]
    iota = lax.broadcasted_iota(jnp.int32, (BLK, 128), 1)
    onehot = (grp == iota).astype(jnp.float32)
    gsum = lax.dot_general(onehot, h, dt, precision=hi,
                           preferred_element_type=jnp.float32)
    cnt = lax.dot_general(onehot, jnp.ones((BLK, 128), jnp.float32), dt,
                          precision=hi, preferred_element_type=jnp.float32)

    @pl.when(i == 0)
    def _():
        gs[...] = jnp.zeros_like(gs)
        cn[...] = jnp.zeros_like(cn)

    gs[...] += gsum
    cn[...] += cnt

    @pl.when(i == NB - 1)
    def _():
        gemb = gs[...][:G, :] / jnp.maximum(cn[...][:G, :], 1.0)
        opt_ref[...] = jax.nn.sigmoid(
            lax.dot_general(gemb, ow[...], (((1,), (0,)), ((), ())),
                            precision=hi,
                            preferred_element_type=jnp.float32) + owb[...])


def kernel(x, edge_index, edge_attr, batch, lin_w, lin_b, fc2_w, fc2_b, fco_w, fco_b):
    row = edge_index[0]
    col = edge_index[1]
    pad_e = E_PAD - E
    rowp = jnp.concatenate([row, jnp.full((pad_e,), N, row.dtype)]).reshape(NCH, CH)
    colp = jnp.concatenate([col, jnp.full((pad_e,), N, col.dtype)]).reshape(NCH, CH)
    eap = jnp.concatenate(
        [edge_attr, jnp.zeros((pad_e,), edge_attr.dtype)]).reshape(NCH, CH)
    xp = jnp.concatenate([x, jnp.zeros((NP - N, D), x.dtype)])
    b2 = jnp.concatenate([batch, jnp.full((NP - N,), 127, batch.dtype)]).reshape(NP, 1)

    deg_kernel, agg_kernel = _sc_kernels()
    cnt2 = deg_kernel(rowp).reshape(NW, R)

    y = pl.pallas_call(
        _prep_body,
        out_shape=jax.ShapeDtypeStruct((NP, D), jnp.float32),
    )(cnt2, xp)

    parts = agg_kernel(y, rowp, colp, eap)
    p0, p1 = parts[0], parts[1]

    ow = jnp.zeros((D, 128), jnp.float32).at[:, 0].set(fco_w[0])
    owb = jnp.zeros((1, 128), jnp.float32).at[0, 0].set(fco_b[0])
    full = lambda shape: pl.BlockSpec(shape, lambda i: (0, 0))
    outp, opt = pl.pallas_call(
        _final_body,
        grid=(NB,),
        in_specs=[
            pl.BlockSpec((BLK, D), lambda i: (i, 0)),
            pl.BlockSpec((BLK, D), lambda i: (i, 0)),
            pl.BlockSpec((NW, BLK), lambda i: (0, i)),
            pl.BlockSpec((BLK, D), lambda i: (i, 0)),
            full((D, D)), full((1, D)),
            full((OUT, D)), full((1, OUT)),
            pl.BlockSpec((BLK, 1), lambda i: (i, 0)),
            full((D, 128)), full((1, 128)),
        ],
        out_specs=(pl.BlockSpec((BLK, OUT), lambda i: (i, 0)),
                   pl.BlockSpec((G, 128), lambda i: (0, 0))),
        out_shape=(jax.ShapeDtypeStruct((NP, OUT), jnp.float32),
                   jax.ShapeDtypeStruct((G, 128), jnp.float32)),
        scratch_shapes=[pltpu.VMEM((128, D), jnp.float32),
                        pltpu.VMEM((128, 128), jnp.float32)],
    )(p0, p1, cnt2, y, lin_w, lin_b.reshape(1, D), fc2_w,
      fc2_b.reshape(1, OUT), b2, ow, owb)

    return (outp[:N], opt[:, 0:1])
